# baseline (speedup 1.0000x reference)
import functools

import jax
import jax.numpy as jnp
from jax import lax
from jax.experimental import pallas as pl
from jax.experimental.pallas import tpu as pltpu
from jax.experimental.pallas import tpu_sc as plsc

NUM_ACTIONS = 1000
MODEL_DIM = 128
BATCH = 16384
HIST = 50

NC = 2
NS = 16
NW = NC * NS

N_WORDS = BATCH * HIST // 2
R_PER_W = N_WORDS // NW
CH = 128
N_STEPS = R_PER_W // CH


def _make_gather():
    mesh = plsc.VectorSubcoreMesh(core_axis_name="c", subcore_axis_name="s")

    @functools.partial(
        pl.kernel,
        mesh=mesh,
        out_type=jax.ShapeDtypeStruct((2 * N_WORDS, MODEL_DIM), jnp.bfloat16),
        scratch_types=[
            pltpu.VMEM((CH,), jnp.int32),
            pltpu.VMEM((CH,), jnp.int32),
            pltpu.VMEM((CH,), jnp.int32),
            pltpu.VMEM((CH,), jnp.int32),
            pltpu.VMEM((CH, MODEL_DIM), jnp.int32),
            pltpu.VMEM((CH, MODEL_DIM), jnp.int32),
            pltpu.SemaphoreType.DMA,
            pltpu.SemaphoreType.DMA,
            pltpu.SemaphoreType.DMA,
            pltpu.SemaphoreType.DMA,
        ],
    )
    def k(lo_hbm, hi_hbm, ie_hbm, io_hbm, out_hbm,
          ie0, ie1, io0, io1, wb0, wb1, ls0, ls1, hs0, hs1):
        wid = lax.axis_index("s") * NC + lax.axis_index("c")
        row0 = wid * R_PER_W
        out_words = out_hbm.bitcast(jnp.int32)

        ies = (ie0, ie1)
        ios = (io0, io1)
        wbufs = (wb0, wb1)
        lsems = (ls0, ls1)
        hsems = (hs0, hs1)

        def fire_lo(step, b):
            pltpu.sync_copy(ie_hbm.at[pl.ds(row0 + step * CH, CH)], ies[b])
            pltpu.async_copy(lo_hbm.at[ies[b]], wbufs[b], lsems[b])

        def fire_hi(step, b):
            pltpu.sync_copy(io_hbm.at[pl.ds(row0 + step * CH, CH)], ios[b])
            pltpu.async_copy(hi_hbm.at[ios[b]], wbufs[b], hsems[b], add=True)

        def drain(sem, b):
            pltpu.make_async_copy(
                out_words.at[pl.ds(0, CH)], wbufs[b], sem
            ).wait()

        fire_lo(0, 0)
        fire_lo(1, 1)
        drain(ls0, 0)
        fire_hi(0, 0)

        def group(g, carry):
            for b in range(2):
                i = g * 2 + b
                nb = 1 - b
                drain(hsems[b], b)
                pltpu.sync_copy(
                    wbufs[b],
                    out_words.at[pl.ds(row0 + i * CH, CH)],
                )
                @pl.when(i < N_STEPS - 2)
                def _():
                    fire_lo(i + 2, b)
                @pl.when(i < N_STEPS - 1)
                def _():
                    drain(lsems[nb], nb)
                    fire_hi(i + 1, nb)
            return carry

        lax.fori_loop(0, N_STEPS // 2, group, 0)

    return k


_gather = _make_gather()


def kernel(actions, action_tokens):
    bits = jax.lax.bitcast_convert_type(
        action_tokens.astype(jnp.bfloat16), jnp.uint16
    ).astype(jnp.int32)
    lo_t = bits
    hi_t = bits << 16
    acts_t = actions.astype(jnp.int32).T
    idx_e = acts_t[:, 0::2].reshape(-1)
    idx_o = acts_t[:, 1::2].reshape(-1)
    out_flat = _gather(lo_t, hi_t, idx_e, idx_o)
    return out_flat.reshape(HIST, BATCH, MODEL_DIM).transpose(1, 0, 2)


# speedup vs baseline: 1.4886x; 1.4886x over previous
import functools

import jax
import jax.numpy as jnp
from jax import lax
from jax.experimental import pallas as pl
from jax.experimental.pallas import tpu as pltpu
from jax.experimental.pallas import tpu_sc as plsc

NUM_ACTIONS = 1000
MODEL_DIM = 128
BATCH = 16384
HIST = 50

NC = 2
NS = 16
NW = NC * NS

N_WORDS = BATCH * HIST // 2
COLS = BATCH // 2
C_PER_W = COLS // NW
CH = 128
N_STEPS = HIST * (C_PER_W // CH)


def _make_gather():
    mesh = plsc.VectorSubcoreMesh(core_axis_name="c", subcore_axis_name="s")

    @functools.partial(
        pl.kernel,
        mesh=mesh,
        compiler_params=pltpu.CompilerParams(needs_layout_passes=False),
        out_type=jax.ShapeDtypeStruct((2 * N_WORDS, MODEL_DIM), jnp.bfloat16),
        scratch_types=[
            pltpu.VMEM((2 * CH,), jnp.int32),
            pltpu.VMEM((2 * CH,), jnp.int32),
            pltpu.VMEM((CH,), jnp.int32),
            pltpu.VMEM((CH,), jnp.int32),
            pltpu.VMEM((CH,), jnp.int32),
            pltpu.VMEM((CH,), jnp.int32),
            pltpu.VMEM((CH, MODEL_DIM), jnp.int32),
            pltpu.VMEM((CH, MODEL_DIM), jnp.int32),
            pltpu.SemaphoreType.DMA,
            pltpu.SemaphoreType.DMA,
            pltpu.SemaphoreType.DMA,
            pltpu.SemaphoreType.DMA,
        ],
    )
    def k(lo_hbm, hi_hbm, act_hbm, out_hbm,
          wi0, wi1, ie0, ie1, io0, io1, wb0, wb1, ls0, ls1, hs0, hs1):
        wid = lax.axis_index("s") * NC + lax.axis_index("c")
        col0 = wid * C_PER_W
        out_words = out_hbm.bitcast(jnp.int32)

        evens = lax.iota(jnp.int32, 16) * 2

        wins = (wi0, wi1)
        ies = (ie0, ie1)
        ios = (io0, io1)
        wbufs = (wb0, wb1)
        lsems = (ls0, ls1)
        hsems = (hs0, hs1)

        def coords(step):
            t = step // (C_PER_W // CH)
            c = col0 + (step % (C_PER_W // CH)) * CH
            return t, c, t * COLS + c

        def fire_lo(step, b):
            t, c, _ = coords(step)
            pltpu.sync_copy(act_hbm.at[t, pl.ds(2 * c, 2 * CH)], wins[b])
            for j in range(CH // 16):
                pos = evens + (32 * j)
                ies[b][pl.ds(16 * j, 16)] = plsc.load_gather(wins[b], [pos])
                ios[b][pl.ds(16 * j, 16)] = plsc.load_gather(wins[b], [pos + 1])
            pltpu.async_copy(lo_hbm.at[ies[b]], wbufs[b], lsems[b])

        def fire_hi(step, b):
            pltpu.async_copy(hi_hbm.at[ios[b]], wbufs[b], hsems[b], add=True)

        def drain(sem, b):
            pltpu.make_async_copy(
                out_words.at[pl.ds(0, CH)], wbufs[b], sem
            ).wait()

        fire_lo(0, 0)
        fire_lo(1, 1)
        drain(ls0, 0)
        fire_hi(0, 0)

        def group(g, carry):
            for b in range(2):
                i = g * 2 + b
                nb = 1 - b
                drain(hsems[b], b)
                pltpu.sync_copy(
                    wbufs[b],
                    out_words.at[pl.ds(coords(i)[2], CH)],
                )
                @pl.when(i < N_STEPS - 2)
                def _():
                    fire_lo(i + 2, b)
                @pl.when(i < N_STEPS - 1)
                def _():
                    drain(lsems[nb], nb)
                    fire_hi(i + 1, nb)
            return carry

        lax.fori_loop(0, N_STEPS // 2, group, 0)

    return k


_gather = _make_gather()


def kernel(actions, action_tokens):
    bits = jax.lax.bitcast_convert_type(
        action_tokens.astype(jnp.bfloat16), jnp.uint16
    ).astype(jnp.int32)
    lo_t = bits
    hi_t = bits << 16
    acts_t = actions.astype(jnp.int32).T
    out_flat = _gather(lo_t, hi_t, acts_t)
    return out_flat.reshape(HIST, BATCH, MODEL_DIM).transpose(1, 0, 2)


# speedup vs baseline: 1.6723x; 1.1234x over previous
import functools

import jax
import jax.numpy as jnp
from jax import lax
from jax.experimental import pallas as pl
from jax.experimental.pallas import tpu as pltpu
from jax.experimental.pallas import tpu_sc as plsc

NUM_ACTIONS = 1000
MODEL_DIM = 128
BATCH = 16384
HIST = 50

NC = 2
NS = 16
NW = NC * NS

N_WORDS = BATCH * HIST // 2
COLS = BATCH // 2
C_PER_W = COLS // NW
CH = 256
N_STEPS = HIST * (C_PER_W // CH)


def _make_gather():
    mesh = plsc.VectorSubcoreMesh(core_axis_name="c", subcore_axis_name="s")

    @functools.partial(
        pl.kernel,
        mesh=mesh,
        compiler_params=pltpu.CompilerParams(needs_layout_passes=False),
        out_type=jax.ShapeDtypeStruct((2 * N_WORDS, MODEL_DIM), jnp.bfloat16),
        scratch_types=[
            pltpu.VMEM((2 * CH,), jnp.int32),
            pltpu.VMEM((2 * CH,), jnp.int32),
            pltpu.VMEM((CH,), jnp.int32),
            pltpu.VMEM((CH,), jnp.int32),
            pltpu.VMEM((CH,), jnp.int32),
            pltpu.VMEM((CH,), jnp.int32),
            pltpu.VMEM((CH, MODEL_DIM), jnp.int32),
            pltpu.VMEM((CH, MODEL_DIM), jnp.int32),
            pltpu.SemaphoreType.DMA,
            pltpu.SemaphoreType.DMA,
            pltpu.SemaphoreType.DMA,
            pltpu.SemaphoreType.DMA,
        ],
    )
    def k(lo_hbm, hi_hbm, act_hbm, out_hbm,
          wi0, wi1, ie0, ie1, io0, io1, wb0, wb1, ls0, ls1, hs0, hs1):
        wid = lax.axis_index("s") * NC + lax.axis_index("c")
        col0 = wid * C_PER_W
        out_words = out_hbm.bitcast(jnp.int32)

        evens = lax.iota(jnp.int32, 16) * 2

        wins = (wi0, wi1)
        ies = (ie0, ie1)
        ios = (io0, io1)
        wbufs = (wb0, wb1)
        lsems = (ls0, ls1)
        hsems = (hs0, hs1)

        def coords(step):
            t = step // (C_PER_W // CH)
            c = col0 + (step % (C_PER_W // CH)) * CH
            return t, c, t * COLS + c

        def fire_lo(step, b):
            t, c, _ = coords(step)
            pltpu.sync_copy(act_hbm.at[t, pl.ds(2 * c, 2 * CH)], wins[b])
            for j in range(CH // 16):
                pos = evens + (32 * j)
                ies[b][pl.ds(16 * j, 16)] = plsc.load_gather(wins[b], [pos])
                ios[b][pl.ds(16 * j, 16)] = plsc.load_gather(wins[b], [pos + 1])
            pltpu.async_copy(lo_hbm.at[ies[b]], wbufs[b], lsems[b])

        def fire_hi(step, b):
            pltpu.async_copy(hi_hbm.at[ios[b]], wbufs[b], hsems[b], add=True)

        def drain(sem, b):
            pltpu.make_async_copy(
                out_words.at[pl.ds(0, CH)], wbufs[b], sem
            ).wait()

        fire_lo(0, 0)
        fire_lo(1, 1)
        drain(ls0, 0)
        fire_hi(0, 0)

        def group(g, carry):
            for b in range(2):
                i = g * 2 + b
                nb = 1 - b
                drain(hsems[b], b)
                pltpu.sync_copy(
                    wbufs[b],
                    out_words.at[pl.ds(coords(i)[2], CH)],
                )
                @pl.when(i < N_STEPS - 2)
                def _():
                    fire_lo(i + 2, b)
                @pl.when(i < N_STEPS - 1)
                def _():
                    drain(lsems[nb], nb)
                    fire_hi(i + 1, nb)
            return carry

        lax.fori_loop(0, N_STEPS // 2, group, 0)

    return k


_gather = _make_gather()


def kernel(actions, action_tokens):
    bits = jax.lax.bitcast_convert_type(
        action_tokens.astype(jnp.bfloat16), jnp.uint16
    ).astype(jnp.int32)
    lo_t = bits
    hi_t = bits << 16
    acts_t = actions.astype(jnp.int32).T
    out_flat = _gather(lo_t, hi_t, acts_t)
    return out_flat.reshape(HIST, BATCH, MODEL_DIM).transpose(1, 0, 2)
